# baseline (speedup 1.0000x reference)
import jax
import jax.numpy as jnp
from jax.experimental import pallas as pl
from jax.experimental.pallas import tpu as pltpu

_LANES = 512
_GRID = 2


def _l1_mean_kernel(inv_n_ref, p_ref, g_ref, o_ref, acc_ref):
    i = pl.program_id(0)

    @pl.when(i == 0)
    def _init():
        acc_ref[...] = jnp.zeros_like(acc_ref)

    acc_ref[...] += jnp.sum(jnp.abs(p_ref[...] - g_ref[...]), axis=0,
                            keepdims=True)

    @pl.when(i == pl.num_programs(0) - 1)
    def _finish():
        o_ref[0, 0] = jnp.sum(acc_ref[...]) * inv_n_ref[0]


def kernel(pred_logits, gt_sdf):
    p = pred_logits.reshape(-1, _LANES)
    g = gt_sdf.reshape(-1, _LANES)
    rows = p.shape[0]
    blk = rows // _GRID
    inv_n = jnp.full((1,), 1.0 / p.size, dtype=jnp.float32)
    total = pl.pallas_call(
        _l1_mean_kernel,
        grid=(_GRID,),
        in_specs=[
            pl.BlockSpec(memory_space=pltpu.SMEM),
            pl.BlockSpec((blk, _LANES), lambda i: (i, 0)),
            pl.BlockSpec((blk, _LANES), lambda i: (i, 0)),
        ],
        out_specs=pl.BlockSpec(memory_space=pltpu.SMEM),
        out_shape=jax.ShapeDtypeStruct((1, 1), jnp.float32),
        scratch_shapes=[pltpu.VMEM((1, _LANES), jnp.float32)],
    )(inv_n, p, g)
    return total[0, 0]


# speedup vs baseline: 1.0503x; 1.0503x over previous
import jax
import jax.numpy as jnp
from jax.experimental import pallas as pl
from jax.experimental.pallas import tpu as pltpu

_LANES = 512
_GRID = 4


def _l1_mean_kernel(inv_n_ref, p_ref, g_ref, o_ref, acc_ref):
    i = pl.program_id(0)

    @pl.when(i == 0)
    def _init():
        acc_ref[...] = jnp.zeros_like(acc_ref)

    acc_ref[...] += jnp.sum(jnp.abs(p_ref[...] - g_ref[...]), axis=0,
                            keepdims=True)

    @pl.when(i == pl.num_programs(0) - 1)
    def _finish():
        o_ref[0, 0] = jnp.sum(acc_ref[...]) * inv_n_ref[0]


def kernel(pred_logits, gt_sdf):
    p = pred_logits.reshape(-1, _LANES)
    g = gt_sdf.reshape(-1, _LANES)
    rows = p.shape[0]
    blk = rows // _GRID
    inv_n = jnp.full((1,), 1.0 / p.size, dtype=jnp.float32)
    total = pl.pallas_call(
        _l1_mean_kernel,
        grid=(_GRID,),
        in_specs=[
            pl.BlockSpec(memory_space=pltpu.SMEM),
            pl.BlockSpec((blk, _LANES), lambda i: (i, 0)),
            pl.BlockSpec((blk, _LANES), lambda i: (i, 0)),
        ],
        out_specs=pl.BlockSpec(memory_space=pltpu.SMEM),
        out_shape=jax.ShapeDtypeStruct((1, 1), jnp.float32),
        scratch_shapes=[pltpu.VMEM((1, _LANES), jnp.float32)],
    )(inv_n, p, g)
    return total[0, 0]


# speedup vs baseline: 1.0627x; 1.0119x over previous
import jax
import jax.numpy as jnp
from jax.experimental import pallas as pl
from jax.experimental.pallas import tpu as pltpu

_LANES = 512
_GRID = 4


def _l1_mean_kernel(inv_n_ref, p0_ref, p1_ref, g0_ref, g1_ref, o_ref,
                    acc_ref):
    i = pl.program_id(0)

    @pl.when(i == 0)
    def _init():
        acc_ref[...] = jnp.zeros_like(acc_ref)

    part = (jnp.sum(jnp.abs(p0_ref[...] - g0_ref[...]), axis=0, keepdims=True)
            + jnp.sum(jnp.abs(p1_ref[...] - g1_ref[...]), axis=0,
                      keepdims=True))
    acc_ref[...] += part

    @pl.when(i == pl.num_programs(0) - 1)
    def _finish():
        o_ref[0, 0] = jnp.sum(acc_ref[...]) * inv_n_ref[0]


def kernel(pred_logits, gt_sdf):
    p = pred_logits.reshape(-1, _LANES)
    g = gt_sdf.reshape(-1, _LANES)
    rows = p.shape[0]
    blk = rows // (2 * _GRID)
    inv_n = jnp.full((1,), 1.0 / p.size, dtype=jnp.float32)
    lo = pl.BlockSpec((blk, _LANES), lambda i: (i, 0))
    hi = pl.BlockSpec((blk, _LANES), lambda i: (i + _GRID, 0))
    total = pl.pallas_call(
        _l1_mean_kernel,
        grid=(_GRID,),
        in_specs=[pl.BlockSpec(memory_space=pltpu.SMEM), lo, hi, lo, hi],
        out_specs=pl.BlockSpec(memory_space=pltpu.SMEM),
        out_shape=jax.ShapeDtypeStruct((1, 1), jnp.float32),
        scratch_shapes=[pltpu.VMEM((1, _LANES), jnp.float32)],
    )(inv_n, p, p, g, g)
    return total[0, 0]
